# baseline (speedup 1.0000x reference)
import functools

import jax
import jax.numpy as jnp
from jax import lax
from jax.experimental import pallas as pl
from jax.experimental.pallas import tpu as pltpu
from jax.experimental.pallas import tpu_sc as plsc

_D = 128
_B = 16384
_ALPHA = 0.1
_NC, _NS, _L = 1, 16, 16
_NW = _NC * _NS
_BPW = _B // _NW
_C = 128
_NCHUNK = _BPW // _C


def _sqrt16(x):
    i = lax.bitcast_convert_type(x, jnp.int32)
    y = lax.bitcast_convert_type(
        jnp.int32(0x5F3759DF) - lax.shift_right_logical(i, 1), jnp.float32)
    half = x * jnp.float32(0.5)
    for _ in range(3):
        y = y * (jnp.float32(1.5) - half * y * y)
    return x * y


def _sc_body(h_hbm, idx_hbm, t_hbm, rel_hbm, out_hbm,
             idx0, idx1, r0, r1, h0, h1, t0, t1, pd_v, pt_v, o_v,
             sr0, sr1, sh0, sh1, st0, st1):
    wid = lax.axis_index("s") * _NC + lax.axis_index("c")
    lanes = lax.iota(jnp.int32, 16)
    zero16 = jnp.zeros((16,), jnp.float32)
    idxs = (idx0, idx1)
    rb, hb, tb = (r0, r1), (h0, h1), (t0, t1)
    sr, sh, st = (sr0, sr1), (sh0, sh1), (st0, st1)

    def start(g, bi):
        base = wid * _BPW + g * _C
        pltpu.sync_copy(idx_hbm.at[pl.ds(base, _C)], idxs[bi])
        return (
            pltpu.async_copy(rel_hbm.at[idxs[bi]], rb[bi], sr[bi]),
            pltpu.async_copy(h_hbm.at[pl.ds(base, _C)], hb[bi], sh[bi]),
            pltpu.async_copy(t_hbm.at[pl.ds(base, _C)], tb[bi], st[bi]),
        )

    cps = [start(0, 0), None]
    for g in range(_NCHUNK):
        bi = g % 2
        if g + 1 < _NCHUNK:
            cps[1 - bi] = start(g + 1, 1 - bi)
        for cp in cps[bi]:
            cp.wait()
        r_v, h_v, t_v = rb[bi], hb[bi], tb[bi]

        scat = lanes * jnp.int32(_C + 1)

        def row_body(i, carry):
            acc_d = zero16
            acc_t = zero16
            for j in range(_D // _L):
                s = pl.ds(j * _L, _L)
                hh = h_v[i, s]
                rr = r_v[i, s]
                tt = t_v[i, s]
                acc_d = acc_d + hh * rr * tt
                diff = (hh + rr) - tt
                acc_t = acc_t + diff * diff
            pos = scat + i
            plsc.store_scatter(pd_v, [pos], acc_d)
            plsc.store_scatter(pt_v, [pos], acc_t)
            return carry

        lax.fori_loop(0, _C, row_body, jnp.int32(0), unroll=4)

        for blk in range(_C // _L):
            acc_d = zero16
            acc_t = zero16
            for j in range(_L):
                s = pl.ds(j * (_C + 1) + blk * _L, _L)
                acc_d = acc_d + pd_v[s]
                acc_t = acc_t + pt_v[s]
            o_v[pl.ds(blk * _L, _L)] = acc_d - jnp.float32(_ALPHA) * _sqrt16(acc_t)
        pltpu.sync_copy(o_v, out_hbm.at[pl.ds(wid * _BPW + g * _C, _C)])


@jax.jit
def _impl(h_emb, r_idx, t_emb, rel_emb):
    mesh = plsc.VectorSubcoreMesh(core_axis_name="c", subcore_axis_name="s",
                                  num_cores=_NC)
    k = pl.kernel(
        _sc_body,
        mesh=mesh,
        compiler_params=pltpu.CompilerParams(needs_layout_passes=False),
        out_type=jax.ShapeDtypeStruct((_B,), jnp.float32),
        scratch_types=(
            [pltpu.VMEM((_C,), jnp.int32)] * 2
            + [pltpu.VMEM((_C, _D), jnp.float32)] * 6
            + [pltpu.VMEM((_L * (_C + 1),), jnp.float32)] * 2
            + [pltpu.VMEM((_C,), jnp.float32)]
            + [pltpu.SemaphoreType.DMA] * 6
        ),
    )
    return k(h_emb, r_idx.astype(jnp.int32), t_emb, rel_emb)


def kernel(h_emb, r_idx, t_emb, rel_emb):
    return _impl(h_emb, r_idx, t_emb, rel_emb)


# speedup vs baseline: 1.4293x; 1.4293x over previous
import functools

import jax
import jax.numpy as jnp
from jax import lax
from jax.experimental import pallas as pl
from jax.experimental.pallas import tpu as pltpu
from jax.experimental.pallas import tpu_sc as plsc

_D = 128
_B = 16384
_ALPHA = 0.1
_NC, _NS, _L = 2, 16, 16
_NW = _NC * _NS
_BPW = _B // _NW
_C = 128
_NCHUNK = _BPW // _C


def _sqrt16(x):
    i = lax.bitcast_convert_type(x, jnp.int32)
    y = lax.bitcast_convert_type(
        jnp.int32(0x5F3759DF) - lax.shift_right_logical(i, 1), jnp.float32)
    half = x * jnp.float32(0.5)
    for _ in range(3):
        y = y * (jnp.float32(1.5) - half * y * y)
    return x * y


def _sc_body(h_hbm, idx_hbm, t_hbm, rel_hbm, out_hbm,
             idx2, r0, r1, h0, h1, t0, t1, pd_v, pt_v, o_v,
             si, sr0, sr1, sh0, sh1, st0, st1):
    wid = lax.axis_index("s") * _NC + lax.axis_index("c")
    lanes = lax.iota(jnp.int32, 16)
    zero16 = jnp.zeros((16,), jnp.float32)
    rb, hb, tb = (r0, r1), (h0, h1), (t0, t1)
    sr, sh, st = (sr0, sr1), (sh0, sh1), (st0, st1)

    def start_ht(g, bi):
        base = wid * _BPW + g * _C
        return (
            pltpu.async_copy(h_hbm.at[pl.ds(base, _C)], hb[bi], sh[bi]),
            pltpu.async_copy(t_hbm.at[pl.ds(base, _C)], tb[bi], st[bi]),
        )

    def start_r(g, bi):
        return pltpu.async_copy(rel_hbm.at[idx2.at[g]], rb[bi], sr[bi])

    ht0 = start_ht(0, 0)
    idx_cps = [
        pltpu.async_copy(idx_hbm.at[pl.ds(wid * _BPW + g * _C, _C)],
                         idx2.at[g], si)
        for g in range(_NCHUNK)
    ]
    for cp in idx_cps:
        cp.wait()
    cps = [(start_r(0, 0),) + ht0, None]
    for g in range(_NCHUNK):
        bi = g % 2
        if g + 1 < _NCHUNK:
            cps[1 - bi] = (start_r(g + 1, 1 - bi),) + start_ht(g + 1, 1 - bi)
        for cp in cps[bi]:
            cp.wait()
        r_v, h_v, t_v = rb[bi], hb[bi], tb[bi]

        scat = lanes * jnp.int32(_C + 1)

        def row_body(i, carry):
            acc_d = zero16
            acc_t = zero16
            for j in range(_D // _L):
                s = pl.ds(j * _L, _L)
                hh = h_v[i, s]
                rr = r_v[i, s]
                tt = t_v[i, s]
                acc_d = acc_d + hh * rr * tt
                diff = (hh + rr) - tt
                acc_t = acc_t + diff * diff
            pos = scat + i
            plsc.store_scatter(pd_v, [pos], acc_d)
            plsc.store_scatter(pt_v, [pos], acc_t)
            return carry

        lax.fori_loop(0, _C, row_body, jnp.int32(0), unroll=4)

        for blk in range(_C // _L):
            acc_d = zero16
            acc_t = zero16
            for j in range(_L):
                s = pl.ds(j * (_C + 1) + blk * _L, _L)
                acc_d = acc_d + pd_v[s]
                acc_t = acc_t + pt_v[s]
            o_v[pl.ds(g * _C + blk * _L, _L)] = (
                acc_d - jnp.float32(_ALPHA) * _sqrt16(acc_t))
    pltpu.sync_copy(o_v, out_hbm.at[pl.ds(wid * _BPW, _BPW)])


@jax.jit
def _impl(h_emb, r_idx, t_emb, rel_emb):
    mesh = plsc.VectorSubcoreMesh(core_axis_name="c", subcore_axis_name="s",
                                  num_cores=_NC)
    k = pl.kernel(
        _sc_body,
        mesh=mesh,
        compiler_params=pltpu.CompilerParams(needs_layout_passes=False),
        out_type=jax.ShapeDtypeStruct((_B,), jnp.float32),
        scratch_types=(
            [pltpu.VMEM((_NCHUNK, _C), jnp.int32)]
            + [pltpu.VMEM((_C, _D), jnp.float32)] * 6
            + [pltpu.VMEM((_L * (_C + 1),), jnp.float32)] * 2
            + [pltpu.VMEM((_BPW,), jnp.float32)]
            + [pltpu.SemaphoreType.DMA] * 7
        ),
    )
    return k(h_emb, r_idx.astype(jnp.int32), t_emb, rel_emb)


def kernel(h_emb, r_idx, t_emb, rel_emb):
    return _impl(h_emb, r_idx, t_emb, rel_emb)
